# Initial kernel scaffold; baseline (speedup 1.0000x reference)
import jax
import jax.numpy as jnp
from jax.experimental import pallas as pl


def kernel(x, W1, b1, W2, b2, W3, b3, E, W4, b4, W5, b5, W6, b6):
    raise NotImplementedError("write your pallas kernel here")


# speedup vs baseline: 1.1519x; 1.1519x over previous
import functools

import jax
import jax.numpy as jnp
from jax import lax
from jax.experimental import pallas as pl
from jax.experimental.pallas import tpu as pltpu
from jax.experimental.pallas import tpu_sc as plsc

_B, _D, _LAT, _K = 4096, 512, 256, 8192
_H1, _H2 = 512, 256
_BM = 512
_NK = 2048


def _encvq_body(x_ref, w1_ref, b1_ref, w2_ref, b2_ref, w3_ref, b3_ref, e_ref,
                ze_ref, idx_ref):
    x = x_ref[...]
    h = jnp.maximum(
        jnp.dot(x, w1_ref[...], preferred_element_type=jnp.float32) + b1_ref[...], 0.0)
    h = jnp.maximum(
        jnp.dot(h, w2_ref[...], preferred_element_type=jnp.float32) + b2_ref[...], 0.0)
    z = jnp.dot(h, w3_ref[...], preferred_element_type=jnp.float32) + b3_ref[...]
    ze_ref[...] = z
    z2 = z * z
    acc = z2[:, 0:8]
    for t in range(1, _LAT // 8):
        acc = acc + z2[:, 8 * t : 8 * t + 8]
    a4 = acc[:, :4] + acc[:, 4:]
    a2 = a4[:, :2] + a4[:, 2:]
    c = a2[:, 0:1] + a2[:, 1:2]
    best = None
    besti = None
    for t in range(_K // _NK):
        ek = e_ref[pl.ds(t * _NK, _NK), :]
        s = jnp.sum(ek * ek, axis=1)
        p = lax.dot_general(z, ek, (((1,), (1,)), ((), ())),
                            preferred_element_type=jnp.float32)
        d2 = (c + s[None, :]) - 2.0 * p
        dist = jnp.sqrt(jnp.maximum(d2, 0.0))
        m = jnp.min(dist, axis=1, keepdims=True)
        iota = lax.broadcasted_iota(jnp.int32, (_BM, _NK), 1)
        cand = jnp.where(dist == m, iota, _NK)
        a = jnp.min(cand, axis=1, keepdims=True) + t * _NK
        if best is None:
            best, besti = m, a
        else:
            upd = m < best
            best = jnp.where(upd, m, best)
            besti = jnp.where(upd, a, besti)
    idx_ref[...] = besti


def _dec_body(zq_ref, w4_ref, b4_ref, w5_ref, b5_ref, w6_ref, b6_ref, out_ref):
    g = jnp.maximum(
        jnp.dot(zq_ref[...], w4_ref[...], preferred_element_type=jnp.float32) + b4_ref[...], 0.0)
    g = jnp.maximum(
        jnp.dot(g, w5_ref[...], preferred_element_type=jnp.float32) + b5_ref[...], 0.0)
    out_ref[...] = jnp.dot(g, w6_ref[...], preferred_element_type=jnp.float32) + b6_ref[...]


def _vq_gather(e, idx):
    info = plsc.get_sparse_core_info()
    nc, ns = info.num_cores, info.num_subcores
    nw = nc * ns
    bpw = _B // nw
    mesh = plsc.VectorSubcoreMesh(core_axis_name="c", subcore_axis_name="s")

    @functools.partial(
        pl.kernel, mesh=mesh,
        out_type=jax.ShapeDtypeStruct((_B, _LAT), jnp.float32),
        scratch_types=[
            pltpu.VMEM((bpw,), jnp.int32),
            pltpu.VMEM((bpw, _LAT), jnp.float32),
            pltpu.SemaphoreType.DMA,
        ],
    )
    def k(table_hbm, idx_hbm, out_hbm, idx_v, rows_v, sem):
        wid = lax.axis_index("s") * nc + lax.axis_index("c")
        base = wid * bpw
        pltpu.sync_copy(idx_hbm.at[pl.ds(base, bpw)], idx_v)
        pltpu.async_copy(table_hbm.at[idx_v], rows_v, sem).wait()
        pltpu.sync_copy(rows_v, out_hbm.at[pl.ds(base, bpw)])

    return k(e, idx)


def _full(shape):
    return pl.BlockSpec(shape, lambda i: tuple(0 for _ in shape))


def kernel(x, W1, b1, W2, b2, W3, b3, E, W4, b4, W5, b5, W6, b6):
    grid = (_B // _BM,)
    ze, idx2 = pl.pallas_call(
        _encvq_body,
        grid=grid,
        in_specs=[
            pl.BlockSpec((_BM, _D), lambda i: (i, 0)),
            _full((_D, _H1)), _full((1, _H1)),
            _full((_H1, _H2)), _full((1, _H2)),
            _full((_H2, _LAT)), _full((1, _LAT)),
            _full((_K, _LAT)),
        ],
        out_specs=[
            pl.BlockSpec((_BM, _LAT), lambda i: (i, 0)),
            pl.BlockSpec((_BM, 1), lambda i: (i, 0)),
        ],
        out_shape=[
            jax.ShapeDtypeStruct((_B, _LAT), jnp.float32),
            jax.ShapeDtypeStruct((_B, 1), jnp.int32),
        ],
    )(x, W1, b1.reshape(1, -1), W2, b2.reshape(1, -1), W3, b3.reshape(1, -1), E)
    indices = idx2.reshape(_B)
    zq = _vq_gather(E, indices)
    xr = pl.pallas_call(
        _dec_body,
        grid=grid,
        in_specs=[
            pl.BlockSpec((_BM, _LAT), lambda i: (i, 0)),
            _full((_LAT, _H2)), _full((1, _H2)),
            _full((_H2, _H1)), _full((1, _H1)),
            _full((_H1, _D)), _full((1, _D)),
        ],
        out_specs=pl.BlockSpec((_BM, _D), lambda i: (i, 0)),
        out_shape=jax.ShapeDtypeStruct((_B, _D), jnp.float32),
    )(zq, W4, b4.reshape(1, -1), W5, b5.reshape(1, -1), W6, b6.reshape(1, -1))
    return (xr, ze, zq, indices)
